# baseline (speedup 1.0000x reference)
import functools

import jax
import jax.numpy as jnp
from jax import lax
from jax.experimental import pallas as pl
from jax.experimental.pallas import tpu as pltpu
from jax.experimental.pallas import tpu_sc as plsc

_W = 64
_D = 64
_B = 16384
_INFO = plsc.get_sparse_core_info()
_NC, _NS, _L = _INFO.num_cores, _INFO.num_subcores, _INFO.num_lanes
_NW = _NC * _NS
_BPW = _B // _NW
_CHUNK = 128
_NCHUNK = _BPW // _CHUNK

_mesh = plsc.VectorSubcoreMesh(core_axis_name="c", subcore_axis_name="s")


@functools.partial(
    pl.kernel,
    mesh=_mesh,
    compiler_params=pltpu.CompilerParams(use_tc_tiling_on_sc=False),
    out_type=jax.ShapeDtypeStruct((_B, _D), jnp.float32),
    scratch_types=[
        pltpu.VMEM((_BPW,), jnp.float32),
        pltpu.VMEM((_BPW,), jnp.float32),
        pltpu.VMEM((_BPW,), jnp.float32),
        pltpu.VMEM((_NCHUNK, _CHUNK), jnp.int32),
        pltpu.VMEM((_NCHUNK, _CHUNK, _D), jnp.float32),
        pltpu.SemaphoreType.DMA,
    ],
)
def _lookup(table_hbm, flt_hbm, out_hbm, f0_v, f1_v, f2_v, idx_v, rows_v, sem):
    wid = lax.axis_index("s") * _NC + lax.axis_index("c")
    base = wid * _BPW

    pltpu.sync_copy(flt_hbm.at[pl.ds(base, _BPW)], f0_v)
    pltpu.sync_copy(flt_hbm.at[pl.ds(_B + base, _BPW)], f1_v)
    pltpu.sync_copy(flt_hbm.at[pl.ds(2 * _B + base, _BPW)], f2_v)

    gathers = []
    for j in range(_NCHUNK):
        for t in range(_CHUNK // _L):
            p = j * _CHUNK + t * _L
            f0 = f0_v[pl.ds(p, _L)]
            f1 = f1_v[pl.ds(p, _L)]
            f2 = f2_v[pl.ds(p, _L)]
            r = ((f0 * float(_W)).astype(jnp.int32) * (_W * _W)
                 + (f1 * float(_W)).astype(jnp.int32) * _W
                 + (f2 * float(_W)).astype(jnp.int32))
            idx_v[j, pl.ds(t * _L, _L)] = r
        gathers.append(pltpu.async_copy(table_hbm.at[idx_v.at[j]], rows_v.at[j], sem))
    for cp in gathers:
        cp.wait()
    for j in range(_NCHUNK):
        pltpu.sync_copy(rows_v.at[j], out_hbm.at[pl.ds(base + j * _CHUNK, _CHUNK)])


def kernel(indices, table):
    tbl = table.reshape(_W * _W * _W, _D)
    flt = indices.T.reshape(-1)
    return _lookup(tbl, flt)
